# baseline (speedup 1.0000x reference)
import functools

import jax
import jax.numpy as jnp
from jax import lax
from jax.experimental import pallas as pl
from jax.experimental.pallas import tpu as pltpu
from jax.experimental.pallas import tpu_sc as plsc

D = 128
H = 256
T = 8
OUT = T * D


def _make_gather(Bc: int):
    info = plsc.get_sparse_core_info()
    NC, NS = info.num_cores, info.num_subcores
    NW = NC * NS
    assert Bc % (8 * NW) == 0
    b_per_w = Bc // NW
    mesh = plsc.VectorSubcoreMesh(core_axis_name="c", subcore_axis_name="s")

    @functools.partial(
        pl.kernel,
        mesh=mesh,
        out_type=jax.ShapeDtypeStruct((Bc, D), jnp.float32),
        scratch_types=[
            pltpu.VMEM((b_per_w,), jnp.int32),
            pltpu.VMEM((b_per_w, D), jnp.float32),
            pltpu.SemaphoreType.DMA,
        ],
    )
    def gather(table_hbm, idx_hbm, out_hbm, idx_v, rows_v, sem):
        wid = lax.axis_index("s") * NC + lax.axis_index("c")
        base = wid * b_per_w
        pltpu.sync_copy(idx_hbm.at[pl.ds(base, b_per_w)], idx_v)
        pltpu.async_copy(table_hbm.at[idx_v], rows_v, sem).wait()
        pltpu.sync_copy(rows_v, out_hbm.at[pl.ds(base, b_per_w)])

    return gather


def _mlp_body(x_ref, w1_ref, b1_ref, w2_ref, b2_ref, o_ref):
    x = x_ref[...].astype(jnp.bfloat16)
    h = jnp.dot(x, w1_ref[...], preferred_element_type=jnp.float32) + b1_ref[...]
    h = h * jax.nn.sigmoid(h)
    out = jnp.dot(
        h.astype(jnp.bfloat16), w2_ref[...], preferred_element_type=jnp.float32
    )
    bb = out.shape[0]
    o_ref[...] = out.reshape(bb, T, D) + b2_ref[...][None]


def _mlp_body_aliased(x_ref, w1_ref, b1_ref, w2_ref, b2_ref, prev_ref, o_ref):
    del prev_ref
    _mlp_body(x_ref, w1_ref, b1_ref, w2_ref, b2_ref, o_ref)


def _mlp_chunk(x, W1b, b1, W2b, b2pos, B, block_b, block_off, prev=None):
    Bc = x.shape[0]
    grid = (Bc // block_b,)
    in_specs = [
        pl.BlockSpec((block_b, D), lambda i: (i, 0)),
        pl.BlockSpec((D, H), lambda i: (0, 0)),
        pl.BlockSpec((1, H), lambda i: (0, 0)),
        pl.BlockSpec((H, OUT), lambda i: (0, 0)),
        pl.BlockSpec((T, D), lambda i: (0, 0)),
    ]
    inputs = [x, W1b, b1, W2b, b2pos]
    body = _mlp_body
    kwargs = {}
    if prev is not None:
        in_specs.append(pl.BlockSpec(memory_space=pl.ANY))
        inputs.append(prev)
        body = _mlp_body_aliased
        kwargs["input_output_aliases"] = {5: 0}
    return pl.pallas_call(
        body,
        grid=grid,
        in_specs=in_specs,
        out_specs=pl.BlockSpec(
            (block_b, T, D), lambda i: (i + block_off, 0, 0)
        ),
        out_shape=jax.ShapeDtypeStruct((B, T, D), jnp.float32),
        **kwargs,
    )(*inputs)


def kernel(labels, table, W1, b1, W2, b2, pos):
    B = labels.shape[0]
    idx = labels.astype(jnp.int32)
    W1b = W1.astype(jnp.bfloat16)
    W2b = W2.astype(jnp.bfloat16)
    b1r = b1.reshape(1, H)
    b2pos = (b2 + pos.reshape(OUT)).reshape(T, D)

    n_chunks = 2
    block_b = 2048
    Bc = B // n_chunks
    gather = _make_gather(Bc)
    xs = [gather(table, lax.slice(idx, (c * Bc,), ((c + 1) * Bc,)))
          for c in range(n_chunks)]
    out = None
    for c in range(n_chunks):
        out = _mlp_chunk(
            xs[c], W1b, b1r, W2b, b2pos, B, block_b,
            block_off=c * (Bc // block_b), prev=out,
        )
    return out


# speedup vs baseline: 1.0114x; 1.0114x over previous
import functools

import jax
import jax.numpy as jnp
from jax import lax
from jax.experimental import pallas as pl
from jax.experimental.pallas import tpu as pltpu
from jax.experimental.pallas import tpu_sc as plsc

D = 128
H = 256
T = 8
OUT = T * D


def _make_gather(Bc: int):
    info = plsc.get_sparse_core_info()
    NC, NS = info.num_cores, info.num_subcores
    NW = NC * NS
    assert Bc % (8 * NW) == 0
    b_per_w = Bc // NW
    mesh = plsc.VectorSubcoreMesh(core_axis_name="c", subcore_axis_name="s")

    @functools.partial(
        pl.kernel,
        mesh=mesh,
        out_type=jax.ShapeDtypeStruct((Bc, D), jnp.float32),
        scratch_types=[
            pltpu.VMEM((b_per_w,), jnp.int32),
            pltpu.VMEM((b_per_w, D), jnp.float32),
            pltpu.SemaphoreType.DMA,
        ],
    )
    def gather(table_hbm, idx_hbm, out_hbm, idx_v, rows_v, sem):
        wid = lax.axis_index("s") * NC + lax.axis_index("c")
        base = wid * b_per_w
        pltpu.sync_copy(idx_hbm.at[pl.ds(base, b_per_w)], idx_v)
        pltpu.async_copy(table_hbm.at[idx_v], rows_v, sem).wait()
        pltpu.sync_copy(rows_v, out_hbm.at[pl.ds(base, b_per_w)])

    return gather


def _mlp_body(x_ref, w1_ref, b1_ref, w2_ref, b2_ref, o_ref):
    x = x_ref[...].astype(jnp.bfloat16)
    h = jnp.dot(x, w1_ref[...], preferred_element_type=jnp.float32) + b1_ref[...]
    h = h * jax.nn.sigmoid(h)
    out = jnp.dot(
        h.astype(jnp.bfloat16), w2_ref[...], preferred_element_type=jnp.float32
    )
    bb = out.shape[0]
    o_ref[...] = out.reshape(bb, T, D) + b2_ref[...][None]


def _mlp_body_aliased(x_ref, w1_ref, b1_ref, w2_ref, b2_ref, prev_ref, o_ref):
    del prev_ref
    _mlp_body(x_ref, w1_ref, b1_ref, w2_ref, b2_ref, o_ref)


def _mlp_chunk(x, W1b, b1, W2b, b2pos, B, block_b, block_off, prev=None):
    Bc = x.shape[0]
    grid = (Bc // block_b,)
    in_specs = [
        pl.BlockSpec((block_b, D), lambda i: (i, 0)),
        pl.BlockSpec((D, H), lambda i: (0, 0)),
        pl.BlockSpec((1, H), lambda i: (0, 0)),
        pl.BlockSpec((H, OUT), lambda i: (0, 0)),
        pl.BlockSpec((T, D), lambda i: (0, 0)),
    ]
    inputs = [x, W1b, b1, W2b, b2pos]
    body = _mlp_body
    kwargs = {}
    if prev is not None:
        in_specs.append(pl.BlockSpec(memory_space=pl.ANY))
        inputs.append(prev)
        body = _mlp_body_aliased
        kwargs["input_output_aliases"] = {5: 0}
    return pl.pallas_call(
        body,
        grid=grid,
        in_specs=in_specs,
        out_specs=pl.BlockSpec(
            (block_b, T, D), lambda i: (i + block_off, 0, 0)
        ),
        out_shape=jax.ShapeDtypeStruct((B, T, D), jnp.float32),
        **kwargs,
    )(*inputs)


def kernel(labels, table, W1, b1, W2, b2, pos):
    B = labels.shape[0]
    idx = labels.astype(jnp.int32)
    W1b = W1.astype(jnp.bfloat16)
    W2b = W2.astype(jnp.bfloat16)
    b1r = b1.reshape(1, H)
    b2pos = (b2 + pos.reshape(OUT)).reshape(T, D)

    block_b = 2048
    chunk_sizes = (4096, B - 4096)
    starts = (0, 4096)
    xs = [
        _make_gather(bc)(table, lax.slice(idx, (s,), (s + bc,)))
        for s, bc in zip(starts, chunk_sizes)
    ]
    out = None
    for x, s in zip(xs, starts):
        out = _mlp_chunk(
            x, W1b, b1r, W2b, b2pos, B, block_b,
            block_off=s // block_b, prev=out,
        )
    return out


# speedup vs baseline: 1.0626x; 1.0506x over previous
import functools

import jax
import jax.numpy as jnp
from jax import lax
from jax.experimental import pallas as pl
from jax.experimental.pallas import tpu as pltpu
from jax.experimental.pallas import tpu_sc as plsc

D = 128
H = 256
T = 8
OUT = T * D


def _make_gather(B: int):
    info = plsc.get_sparse_core_info()
    NC, NS = info.num_cores, info.num_subcores
    NW = NC * NS
    assert B % (8 * NW) == 0
    b_per_w = B // NW
    mesh = plsc.VectorSubcoreMesh(core_axis_name="c", subcore_axis_name="s")

    @functools.partial(
        pl.kernel,
        mesh=mesh,
        out_type=jax.ShapeDtypeStruct((B, D), jnp.float32),
        scratch_types=[
            pltpu.VMEM((b_per_w,), jnp.int32),
            pltpu.VMEM((b_per_w, D), jnp.float32),
            pltpu.SemaphoreType.DMA,
        ],
    )
    def gather(table_hbm, idx_hbm, out_hbm, idx_v, rows_v, sem):
        wid = lax.axis_index("s") * NC + lax.axis_index("c")
        base = wid * b_per_w
        pltpu.sync_copy(idx_hbm.at[pl.ds(base, b_per_w)], idx_v)
        pltpu.async_copy(table_hbm.at[idx_v], rows_v, sem).wait()
        pltpu.sync_copy(rows_v, out_hbm.at[pl.ds(base, b_per_w)])

    return gather


def _mlp_body(x_ref, w1_ref, b1_ref, w2_ref, b2_ref, o_ref):
    x = x_ref[...].astype(jnp.bfloat16)
    h = jnp.dot(x, w1_ref[...], preferred_element_type=jnp.float32) + b1_ref[...]
    h = h * jax.nn.sigmoid(h)
    out = jnp.dot(
        h.astype(jnp.bfloat16), w2_ref[...], preferred_element_type=jnp.float32
    )
    bb = out.shape[0]
    o_ref[...] = out.reshape(bb, T, D) + b2_ref[...][None]


def _mlp(x, W1b, b1, W2b, b2pos, block_b: int):
    B = x.shape[0]
    grid = (B // block_b,)
    return pl.pallas_call(
        _mlp_body,
        grid=grid,
        in_specs=[
            pl.BlockSpec((block_b, D), lambda i: (i, 0)),
            pl.BlockSpec((D, H), lambda i: (0, 0)),
            pl.BlockSpec((1, H), lambda i: (0, 0)),
            pl.BlockSpec((H, OUT), lambda i: (0, 0)),
            pl.BlockSpec((T, D), lambda i: (0, 0)),
        ],
        out_specs=pl.BlockSpec((block_b, T, D), lambda i: (i, 0, 0)),
        out_shape=jax.ShapeDtypeStruct((B, T, D), jnp.float32),
    )(x, W1b, b1, W2b, b2pos)


def kernel(labels, table, W1, b1, W2, b2, pos):
    B = labels.shape[0]
    idx = labels.astype(jnp.int32)
    x = _make_gather(B)(table, idx)
    b2pos = (b2 + pos.reshape(OUT)).reshape(T, D)
    return _mlp(
        x,
        W1.astype(jnp.bfloat16),
        b1.reshape(1, H),
        W2.astype(jnp.bfloat16),
        b2pos,
        block_b=2048,
    )
